# baseline (speedup 1.0000x reference)
import functools

import jax
import jax.numpy as jnp
import numpy as np
from jax import lax
from jax.experimental import pallas as pl
from jax.experimental.pallas import tpu as pltpu
from jax.experimental.pallas import tpu_sc as plsc

_FIELD_DIMS = [100000] * 26
_B = 16384
_F = 26
_FP = 32
_D = 16
_DP = 128
_TOTAL = sum(_FIELD_DIMS)
_MLP_IN = _F * _D
_MLP_PAD = _FP * _D

_NC = 2
_NS = 16
_NW = _NC * _NS
_ROWS_W = _B // _NW
_CHUNK = 128
_GROUPS = _ROWS_W // 16
_NCHUNK = 4 * _GROUPS
_NLIN = _F * (_ROWS_W // _CHUNK)


def _sc_gather(xr, xl, emb_pad, lin_flat):
    mesh = plsc.VectorSubcoreMesh(core_axis_name="c", subcore_axis_name="s")

    nb_l = _NLIN // 8

    @functools.partial(
        pl.kernel,
        mesh=mesh,
        out_type=[
            jax.ShapeDtypeStruct((4, _B, _DP), jnp.float32),
            jax.ShapeDtypeStruct((_B,), jnp.float32),
        ],
        scratch_types=[
            pltpu.VMEM((_NCHUNK, _CHUNK), jnp.int32),
            pltpu.VMEM((_NLIN, _CHUNK), jnp.int32),
            pltpu.VMEM((_CHUNK, _DP), jnp.float32),
            pltpu.VMEM((_CHUNK, _DP), jnp.float32),
            pltpu.VMEM((16, _DP), jnp.float32),
            pltpu.VMEM((16, _DP), jnp.float32),
            pltpu.VMEM((8, _CHUNK), jnp.float32),
            pltpu.VMEM((8, _CHUNK), jnp.float32),
            pltpu.VMEM((_ROWS_W,), jnp.float32),
            pltpu.SemaphoreType.DMA,
            pltpu.SemaphoreType.DMA,
            pltpu.SemaphoreType.DMA,
            pltpu.SemaphoreType.DMA,
            pltpu.SemaphoreType.DMA,
            pltpu.SemaphoreType.DMA,
        ],
    )
    def gather_k(xr_hbm, xl_hbm, emb_hbm, lin_hbm, g_out, lin_out,
                 idx_v, lidx_v, rows_a, rows_b, pk_a, pk_b, lbuf_a, lbuf_b, linacc,
                 sem_ea, sem_eb, sem_wa, sem_wb, sem_la, sem_lb):
        wid = lax.axis_index("s") * _NC + lax.axis_index("c")
        pltpu.sync_copy(xr_hbm.at[wid], idx_v)
        pltpu.sync_copy(xl_hbm.at[wid], lidx_v)

        zero16 = jnp.zeros((16,), jnp.float32)

        def zbody(g, carry):
            linacc[pl.ds(g * 16, 16)] = zero16
            return carry

        lax.fori_loop(0, _ROWS_W // 16, zbody, 0)

        def e_dst(c):
            j = c // _GROUPS
            g = lax.rem(c, _GROUPS)
            row0 = (wid * _GROUPS + g) * 16
            return g_out.at[j, pl.ds(row0, 16)]

        def e_fire(buf, sem, c):
            pltpu.async_copy(emb_hbm.at[idx_v.at[c]], buf, sem)

        def e_drain(buf, sem):
            pltpu.make_async_copy(emb_hbm.at[pl.ds(0, _CHUNK)], buf, sem).wait()

        def w_drain(pk, sem):
            pltpu.make_async_copy(g_out.at[0, pl.ds(0, 16)], pk, sem).wait()

        def extract(rows, pk):
            for r in range(16):
                for f in range(8):
                    pk[r, pl.ds(16 * f, 16)] = rows[8 * r + f, pl.ds(0, 16)]

        e_fire(rows_a, sem_ea, 0)

        def ebody(t, carry):
            ca = 2 * t
            cb = 2 * t + 1
            e_fire(rows_b, sem_eb, cb)
            e_drain(rows_a, sem_ea)

            @pl.when(t > 0)
            def _():
                w_drain(pk_a, sem_wa)

            extract(rows_a, pk_a)
            pltpu.async_copy(pk_a, e_dst(ca), sem_wa)

            @pl.when(cb + 1 < _NCHUNK)
            def _():
                e_fire(rows_a, sem_ea, cb + 1)

            e_drain(rows_b, sem_eb)

            @pl.when(t > 0)
            def _():
                w_drain(pk_b, sem_wb)

            extract(rows_b, pk_b)
            pltpu.async_copy(pk_b, e_dst(cb), sem_wb)
            return carry

        lax.fori_loop(0, _NCHUNK // 2, ebody, 0)
        w_drain(pk_a, sem_wa)
        w_drain(pk_b, sem_wb)

        def l_fire(buf, sem, b):
            for i in range(8):
                pltpu.async_copy(lin_hbm.at[lidx_v.at[b * 8 + i]], buf.at[i], sem)

        def l_drain(buf, sem):
            for i in range(8):
                pltpu.make_async_copy(lin_out.at[pl.ds(0, _CHUNK)], buf.at[i], sem).wait()

        def l_reduce(buf):
            for i in range(8):
                for k in range(_CHUNK // 16):
                    off = (i % 4) * _CHUNK + k * 16
                    linacc[pl.ds(off, 16)] = linacc[pl.ds(off, 16)] + buf[i, pl.ds(k * 16, 16)]

        l_fire(lbuf_a, sem_la, 0)

        def lbody(t, carry):
            ba = 2 * t
            bb = 2 * t + 1
            l_fire(lbuf_b, sem_lb, bb)
            l_drain(lbuf_a, sem_la)
            l_reduce(lbuf_a)

            @pl.when(bb + 1 < nb_l)
            def _():
                l_fire(lbuf_a, sem_la, bb + 1)

            l_drain(lbuf_b, sem_lb)
            l_reduce(lbuf_b)
            return carry

        lax.fori_loop(0, nb_l // 2, lbody, 0)
        l_drain(lbuf_a, sem_la)
        l_reduce(lbuf_a)

        pltpu.sync_copy(linacc, lin_out.at[pl.ds(wid * _ROWS_W, _ROWS_W)])

    return gather_k(xr, xl, emb_pad, lin_flat)


_BB = 512


def _tc_body(g_ref, lin_ref, s_ref, w1_ref, b1_ref, w2_ref, b2_ref, w3_ref, cb_ref, out_ref):
    g3 = g_ref[3]
    mask = lax.broadcasted_iota(jnp.int32, g3.shape, 1) < 32
    g3 = jnp.where(mask, g3, 0.0)
    e = jnp.concatenate([g_ref[0], g_ref[1], g_ref[2], g3], axis=1)
    s = jnp.dot(e, s_ref[...], preferred_element_type=jnp.float32)
    fm = 0.5 * (jnp.sum(s * s, axis=1) - jnp.sum(e * e, axis=1))
    fm_logit = jax.nn.sigmoid(fm)
    h = jnp.dot(e, w1_ref[...], preferred_element_type=jnp.float32) + b1_ref[...]
    h = jnp.maximum(h, 0.0)
    h = jnp.dot(h, w2_ref[...], preferred_element_type=jnp.float32) + b2_ref[...]
    h = jnp.maximum(h, 0.0)
    dnn = jnp.dot(h, w3_ref[...], preferred_element_type=jnp.float32)[:, 0]
    logit = lin_ref[...] + fm_logit + dnn + cb_ref[0, 0]
    out_ref[...] = jax.nn.sigmoid(logit)


def _tc_dense(g4, linsum, s_mat, W1p, b1, W2, b2, W3, cb, *, interpret=False):
    grid = (_B // _BB,)
    full = lambda shape: pl.BlockSpec(shape, lambda i: (0,) * len(shape))
    return pl.pallas_call(
        _tc_body,
        grid=grid,
        in_specs=[
            pl.BlockSpec((4, _BB, 128), lambda i: (0, i, 0)),
            pl.BlockSpec((_BB,), lambda i: (i,)),
            full((_MLP_PAD, _D)),
            full((_MLP_PAD, 256)),
            full((1, 256)),
            full((256, 128)),
            full((1, 128)),
            full((128, 1)),
            full((1, 1)),
        ],
        out_specs=pl.BlockSpec((_BB,), lambda i: (i,)),
        out_shape=jax.ShapeDtypeStruct((_B,), jnp.float32),
        interpret=interpret,
    )(g4, linsum, s_mat, W1p, b1, W2, b2, W3, cb)


_OFFSETS = np.cumsum([0] + _FIELD_DIMS[:-1]).astype(np.int32)
_S_MAT = np.concatenate(
    [np.tile(np.eye(_D, dtype=np.float32), (_F, 1)),
     np.zeros((_MLP_PAD - _MLP_IN, _D), np.float32)], axis=0)


def _reorder_indices(x):
    xi = x + _OFFSETS[None, :]
    pad = jnp.zeros((_B, _FP - _F), jnp.int32)
    xp = jnp.concatenate([xi, pad], axis=1)
    xp = xp.reshape(_NW, _GROUPS, 16, 4, 8).transpose(0, 3, 1, 2, 4)
    xr = xp.reshape(_NW, _NCHUNK, _CHUNK)
    xl = xi.reshape(_NW, _ROWS_W // _CHUNK, _CHUNK, _F).transpose(0, 3, 1, 2)
    xl = xl.reshape(_NW, _NLIN, _CHUNK)
    return xr, xl


def kernel(x, emb, lin_w, lin_b, W1, b1, W2, b2, W3, b3):
    xr, xl = _reorder_indices(x)
    emb_pad = jnp.pad(emb, ((0, 0), (0, _DP - _D)))
    g4, linsum = _sc_gather(xr, xl, emb_pad, lin_w.reshape(-1))
    W1p = jnp.concatenate([W1, jnp.zeros((_MLP_PAD - _MLP_IN, 256), jnp.float32)], axis=0)
    cb = (lin_b + b3).reshape(1, 1)
    return _tc_dense(
        g4, linsum, jnp.asarray(_S_MAT), W1p, b1.reshape(1, 256), W2, b2.reshape(1, 128), W3, cb
    )


# speedup vs baseline: 3.7671x; 3.7671x over previous
import functools

import jax
import jax.numpy as jnp
import numpy as np
from jax import lax
from jax.experimental import pallas as pl
from jax.experimental.pallas import tpu as pltpu
from jax.experimental.pallas import tpu_sc as plsc

_FIELD_DIMS = [100000] * 26
_B = 16384
_F = 26
_D = 16
_TOTAL = sum(_FIELD_DIMS)
_MLP_IN = _F * _D
_BF = _B * _F

_NC = 2
_NS = 16
_NW = _NC * _NS
_PER_W = _BF // _NW
_CHUNK = 128
_NCHUNK = _PER_W // _CHUNK


def _sc_gather(xi, emb, lin_flat):
    mesh = plsc.VectorSubcoreMesh(core_axis_name="c", subcore_axis_name="s")

    @functools.partial(
        pl.kernel,
        mesh=mesh,
        compiler_params=pltpu.CompilerParams(use_tc_tiling_on_sc=False),
        out_type=[
            jax.ShapeDtypeStruct((_BF, _D), jnp.float32),
            jax.ShapeDtypeStruct((_BF,), jnp.float32),
        ],
        scratch_types=[
            pltpu.VMEM((_NCHUNK, _CHUNK), jnp.int32),
            pltpu.VMEM((_CHUNK, _D), jnp.float32),
            pltpu.VMEM((_CHUNK,), jnp.float32),
            pltpu.SemaphoreType.DMA,
            pltpu.SemaphoreType.DMA,
        ],
    )
    def gather_k(xi_hbm, emb_hbm, lin_hbm, e_out, lin_out, idx_v, rows_v, lin_v, sem_e, sem_l):
        wid = lax.axis_index("s") * _NC + lax.axis_index("c")
        base = wid * _PER_W
        pltpu.sync_copy(xi_hbm.at[wid], idx_v)

        def body(j, carry):
            cp_e = pltpu.async_copy(emb_hbm.at[idx_v.at[j]], rows_v, sem_e)
            cp_l = pltpu.async_copy(lin_hbm.at[idx_v.at[j]], lin_v, sem_l)
            cp_e.wait()
            cp_l.wait()
            off = base + j * _CHUNK
            pltpu.sync_copy(rows_v, e_out.at[pl.ds(off, _CHUNK)])
            pltpu.sync_copy(lin_v, lin_out.at[pl.ds(off, _CHUNK)])
            return carry

        lax.fori_loop(0, _NCHUNK, body, 0)

    return gather_k(xi, emb, lin_flat)


_BB = 512


def _tc_body(e_ref, lin_ref, s_ref, w1_ref, b1_ref, w2_ref, b2_ref, w3_ref, cb_ref, out_ref):
    e = e_ref[...]
    s = jnp.dot(e, s_ref[...], preferred_element_type=jnp.float32)
    fm = 0.5 * (jnp.sum(s * s, axis=1) - jnp.sum(e * e, axis=1))
    fm_logit = jax.nn.sigmoid(fm)
    lin = jnp.sum(lin_ref[...], axis=1)
    h = jnp.dot(e, w1_ref[...], preferred_element_type=jnp.float32) + b1_ref[...]
    h = jnp.maximum(h, 0.0)
    h = jnp.dot(h, w2_ref[...], preferred_element_type=jnp.float32) + b2_ref[...]
    h = jnp.maximum(h, 0.0)
    dnn = jnp.dot(h, w3_ref[...], preferred_element_type=jnp.float32)[:, 0]
    logit = lin + fm_logit + dnn + cb_ref[0, 0]
    out_ref[...] = jax.nn.sigmoid(logit)


def _tc_dense(e2, linv, s_mat, W1, b1, W2, b2, W3, cb, *, interpret=False):
    grid = (_B // _BB,)
    full = lambda shape: pl.BlockSpec(shape, lambda i: (0,) * len(shape))
    return pl.pallas_call(
        _tc_body,
        grid=grid,
        in_specs=[
            pl.BlockSpec((_BB, _MLP_IN), lambda i: (i, 0)),
            pl.BlockSpec((_BB, _F), lambda i: (i, 0)),
            full((_MLP_IN, _D)),
            full((_MLP_IN, 256)),
            full((1, 256)),
            full((256, 128)),
            full((1, 128)),
            full((128, 1)),
            full((1, 1)),
        ],
        out_specs=pl.BlockSpec((_BB,), lambda i: (i,)),
        out_shape=jax.ShapeDtypeStruct((_B,), jnp.float32),
        interpret=interpret,
    )(e2, linv, s_mat, W1, b1, W2, b2, W3, cb)


_OFFSETS = np.cumsum([0] + _FIELD_DIMS[:-1]).astype(np.int32)
_S_MAT = np.tile(np.eye(_D, dtype=np.float32), (_F, 1))


def kernel(x, emb, lin_w, lin_b, W1, b1, W2, b2, W3, b3):
    xi = (x + _OFFSETS[None, :]).reshape(_NW, _NCHUNK, _CHUNK)
    e_flat, lin_vals = _sc_gather(xi, emb, lin_w.reshape(-1))
    e2 = e_flat.reshape(_B, _MLP_IN)
    linv = lin_vals.reshape(_B, _F)
    cb = (lin_b + b3).reshape(1, 1)
    return _tc_dense(
        e2, linv, jnp.asarray(_S_MAT), W1, b1.reshape(1, 256), W2, b2.reshape(1, 128), W3, cb
    )


# speedup vs baseline: 3.9076x; 1.0373x over previous
import functools

import jax
import jax.numpy as jnp
import numpy as np
from jax import lax
from jax.experimental import pallas as pl
from jax.experimental.pallas import tpu as pltpu
from jax.experimental.pallas import tpu_sc as plsc

_FIELD_DIMS = [100000] * 26
_B = 16384
_F = 26
_D = 16
_TOTAL = sum(_FIELD_DIMS)
_MLP_IN = _F * _D
_BF = _B * _F

_NC = 2
_NS = 16
_NW = _NC * _NS
_PER_W = _BF // _NW
_CHUNK = 128
_NCHUNK = _PER_W // _CHUNK


def _sc_gather(xi, emb, lin_flat):
    mesh = plsc.VectorSubcoreMesh(core_axis_name="c", subcore_axis_name="s")

    @functools.partial(
        pl.kernel,
        mesh=mesh,
        compiler_params=pltpu.CompilerParams(use_tc_tiling_on_sc=False),
        out_type=[
            jax.ShapeDtypeStruct((_BF, _D), jnp.float32),
            jax.ShapeDtypeStruct((_BF,), jnp.float32),
        ],
        scratch_types=[
            pltpu.VMEM((_NCHUNK, _CHUNK), jnp.int32),
            pltpu.VMEM((_CHUNK, _D), jnp.float32),
            pltpu.VMEM((_CHUNK, _D), jnp.float32),
            pltpu.VMEM((_CHUNK,), jnp.float32),
            pltpu.VMEM((_CHUNK,), jnp.float32),
            pltpu.SemaphoreType.DMA,
            pltpu.SemaphoreType.DMA,
            pltpu.SemaphoreType.DMA,
            pltpu.SemaphoreType.DMA,
        ],
    )
    def gather_k(xi_hbm, emb_hbm, lin_hbm, e_out, lin_out, idx_v,
                 rows_a, rows_b, lin_a, lin_b, sem_ea, sem_eb, sem_la, sem_lb):
        wid = lax.axis_index("s") * _NC + lax.axis_index("c")
        base = wid * _PER_W
        pltpu.sync_copy(xi_hbm.at[wid], idx_v)

        def fire(rows, lin, se, sl, j):
            pltpu.async_copy(emb_hbm.at[idx_v.at[j]], rows, se)
            pltpu.async_copy(lin_hbm.at[idx_v.at[j]], lin, sl)

        def drain_write(rows, lin, se, sl, j):
            pltpu.make_async_copy(e_out.at[pl.ds(0, _CHUNK)], rows, se).wait()
            pltpu.make_async_copy(lin_out.at[pl.ds(0, _CHUNK)], lin, sl).wait()
            off = base + j * _CHUNK
            pltpu.sync_copy(rows, e_out.at[pl.ds(off, _CHUNK)])
            pltpu.sync_copy(lin, lin_out.at[pl.ds(off, _CHUNK)])

        fire(rows_a, lin_a, sem_ea, sem_la, 0)

        def body(t, carry):
            ca = 2 * t
            cb = 2 * t + 1
            fire(rows_b, lin_b, sem_eb, sem_lb, cb)
            drain_write(rows_a, lin_a, sem_ea, sem_la, ca)

            @pl.when(cb + 1 < _NCHUNK)
            def _():
                fire(rows_a, lin_a, sem_ea, sem_la, cb + 1)

            drain_write(rows_b, lin_b, sem_eb, sem_lb, cb)
            return carry

        lax.fori_loop(0, _NCHUNK // 2, body, 0)

    return gather_k(xi, emb, lin_flat)


_BB = 512


def _tc_body(e_ref, lin_ref, s_ref, w1_ref, b1_ref, w2_ref, b2_ref, w3_ref, cb_ref, out_ref):
    e = e_ref[...]
    s = jnp.dot(e, s_ref[...], preferred_element_type=jnp.float32)
    fm = 0.5 * (jnp.sum(s * s, axis=1) - jnp.sum(e * e, axis=1))
    fm_logit = jax.nn.sigmoid(fm)
    lin = jnp.sum(lin_ref[...], axis=1)
    h = jnp.dot(e, w1_ref[...], preferred_element_type=jnp.float32) + b1_ref[...]
    h = jnp.maximum(h, 0.0)
    h = jnp.dot(h, w2_ref[...], preferred_element_type=jnp.float32) + b2_ref[...]
    h = jnp.maximum(h, 0.0)
    dnn = jnp.dot(h, w3_ref[...], preferred_element_type=jnp.float32)[:, 0]
    logit = lin + fm_logit + dnn + cb_ref[0, 0]
    out_ref[...] = jax.nn.sigmoid(logit)


def _tc_dense(e2, linv, s_mat, W1, b1, W2, b2, W3, cb, *, interpret=False):
    grid = (_B // _BB,)
    full = lambda shape: pl.BlockSpec(shape, lambda i: (0,) * len(shape))
    return pl.pallas_call(
        _tc_body,
        grid=grid,
        in_specs=[
            pl.BlockSpec((_BB, _MLP_IN), lambda i: (i, 0)),
            pl.BlockSpec((_BB, _F), lambda i: (i, 0)),
            full((_MLP_IN, _D)),
            full((_MLP_IN, 256)),
            full((1, 256)),
            full((256, 128)),
            full((1, 128)),
            full((128, 1)),
            full((1, 1)),
        ],
        out_specs=pl.BlockSpec((_BB,), lambda i: (i,)),
        out_shape=jax.ShapeDtypeStruct((_B,), jnp.float32),
        interpret=interpret,
    )(e2, linv, s_mat, W1, b1, W2, b2, W3, cb)


_OFFSETS = np.cumsum([0] + _FIELD_DIMS[:-1]).astype(np.int32)
_S_MAT = np.tile(np.eye(_D, dtype=np.float32), (_F, 1))


def kernel(x, emb, lin_w, lin_b, W1, b1, W2, b2, W3, b3):
    xi = (x + _OFFSETS[None, :]).reshape(_NW, _NCHUNK, _CHUNK)
    e_flat, lin_vals = _sc_gather(xi, emb, lin_w.reshape(-1))
    e2 = e_flat.reshape(_B, _MLP_IN)
    linv = lin_vals.reshape(_B, _F)
    cb = (lin_b + b3).reshape(1, 1)
    return _tc_dense(
        e2, linv, jnp.asarray(_S_MAT), W1, b1.reshape(1, 256), W2, b2.reshape(1, 128), W3, cb
    )


# speedup vs baseline: 3.9782x; 1.0181x over previous
import functools

import jax
import jax.numpy as jnp
import numpy as np
from jax import lax
from jax.experimental import pallas as pl
from jax.experimental.pallas import tpu as pltpu
from jax.experimental.pallas import tpu_sc as plsc

_FIELD_DIMS = [100000] * 26
_B = 16384
_F = 26
_D = 16
_TOTAL = sum(_FIELD_DIMS)
_MLP_IN = _F * _D
_BF = _B * _F

_NC = 2
_NS = 16
_NW = _NC * _NS
_PER_W = _BF // _NW
_CHUNK = 128
_NCHUNK = _PER_W // _CHUNK


def _sc_gather(xi, emb, lin_flat):
    mesh = plsc.VectorSubcoreMesh(core_axis_name="c", subcore_axis_name="s")

    @functools.partial(
        pl.kernel,
        mesh=mesh,
        compiler_params=pltpu.CompilerParams(use_tc_tiling_on_sc=False),
        out_type=[
            jax.ShapeDtypeStruct((_BF, _D), jnp.float32),
            jax.ShapeDtypeStruct((_BF,), jnp.float32),
        ],
        scratch_types=[
            pltpu.VMEM((_NCHUNK, _CHUNK), jnp.int32),
        ] + [pltpu.VMEM((_CHUNK, _D), jnp.float32)] * 4
          + [pltpu.VMEM((_CHUNK,), jnp.float32)] * 4
          + [pltpu.SemaphoreType.DMA] * 8,
    )
    def gather_k(xi_hbm, emb_hbm, lin_hbm, e_out, lin_out, idx_v,
                 r0, r1, r2, r3, l0, l1, l2, l3,
                 se0, se1, se2, se3, sl0, sl1, sl2, sl3):
        wid = lax.axis_index("s") * _NC + lax.axis_index("c")
        base = wid * _PER_W
        pltpu.sync_copy(xi_hbm.at[wid], idx_v)

        rows = [r0, r1, r2, r3]
        lins = [l0, l1, l2, l3]
        sems_e = [se0, se1, se2, se3]
        sems_l = [sl0, sl1, sl2, sl3]

        def fire(k, j):
            pltpu.async_copy(emb_hbm.at[idx_v.at[j]], rows[k], sems_e[k])
            pltpu.async_copy(lin_hbm.at[idx_v.at[j]], lins[k], sems_l[k])

        def drain_write(k, j):
            pltpu.make_async_copy(e_out.at[pl.ds(0, _CHUNK)], rows[k], sems_e[k]).wait()
            pltpu.make_async_copy(lin_out.at[pl.ds(0, _CHUNK)], lins[k], sems_l[k]).wait()
            off = base + j * _CHUNK
            pltpu.sync_copy(rows[k], e_out.at[pl.ds(off, _CHUNK)])
            pltpu.sync_copy(lins[k], lin_out.at[pl.ds(off, _CHUNK)])

        for k in range(3):
            fire(k, k)

        def body(t, carry):
            for i in range(4):
                c = 4 * t + i

                @pl.when(c + 3 < _NCHUNK)
                def _(c=c, i=i):
                    fire((i + 3) % 4, c + 3)

                drain_write(i, c)
            return carry

        lax.fori_loop(0, _NCHUNK // 4, body, 0)

    return gather_k(xi, emb, lin_flat)


_BB = 512


def _tc_body(e_ref, lin_ref, s_ref, w1_ref, b1_ref, w2_ref, b2_ref, w3_ref, cb_ref, out_ref):
    e = e_ref[...]
    s = jnp.dot(e, s_ref[...], preferred_element_type=jnp.float32)
    fm = 0.5 * (jnp.sum(s * s, axis=1) - jnp.sum(e * e, axis=1))
    fm_logit = jax.nn.sigmoid(fm)
    lin = jnp.sum(lin_ref[...], axis=1)
    h = jnp.dot(e, w1_ref[...], preferred_element_type=jnp.float32) + b1_ref[...]
    h = jnp.maximum(h, 0.0)
    h = jnp.dot(h, w2_ref[...], preferred_element_type=jnp.float32) + b2_ref[...]
    h = jnp.maximum(h, 0.0)
    dnn = jnp.dot(h, w3_ref[...], preferred_element_type=jnp.float32)[:, 0]
    logit = lin + fm_logit + dnn + cb_ref[0, 0]
    out_ref[...] = jax.nn.sigmoid(logit)


def _tc_dense(e2, linv, s_mat, W1, b1, W2, b2, W3, cb, *, interpret=False):
    grid = (_B // _BB,)
    full = lambda shape: pl.BlockSpec(shape, lambda i: (0,) * len(shape))
    return pl.pallas_call(
        _tc_body,
        grid=grid,
        in_specs=[
            pl.BlockSpec((_BB, _MLP_IN), lambda i: (i, 0)),
            pl.BlockSpec((_BB, _F), lambda i: (i, 0)),
            full((_MLP_IN, _D)),
            full((_MLP_IN, 256)),
            full((1, 256)),
            full((256, 128)),
            full((1, 128)),
            full((128, 1)),
            full((1, 1)),
        ],
        out_specs=pl.BlockSpec((_BB,), lambda i: (i,)),
        out_shape=jax.ShapeDtypeStruct((_B,), jnp.float32),
        interpret=interpret,
    )(e2, linv, s_mat, W1, b1, W2, b2, W3, cb)


_OFFSETS = np.cumsum([0] + _FIELD_DIMS[:-1]).astype(np.int32)
_S_MAT = np.tile(np.eye(_D, dtype=np.float32), (_F, 1))


def kernel(x, emb, lin_w, lin_b, W1, b1, W2, b2, W3, b3):
    xi = (x + _OFFSETS[None, :]).reshape(_NW, _NCHUNK, _CHUNK)
    e_flat, lin_vals = _sc_gather(xi, emb, lin_w.reshape(-1))
    e2 = e_flat.reshape(_B, _MLP_IN)
    linv = lin_vals.reshape(_B, _F)
    cb = (lin_b + b3).reshape(1, 1)
    return _tc_dense(
        e2, linv, jnp.asarray(_S_MAT), W1, b1.reshape(1, 256), W2, b2.reshape(1, 128), W3, cb
    )
